# baseline (speedup 1.0000x reference)
import functools

import numpy as np
import jax
import jax.numpy as jnp
from jax import lax
from jax.experimental import pallas as pl
from jax.experimental.pallas import tpu as pltpu
from jax.experimental.pallas import tpu_sc as plsc

_N = 50000
_E = 800000
_S = 64
_V3 = 48
_R = 32
_CUT = 10.0
_W = 128

_NC = 2
_NS = 16
_NW = _NC * _NS

_CH = 128
_NCH = _E // _CH
_GQ, _GR = divmod(_NCH, _NW)
_SQ, _SR = divmod(_NCH, _NS)
_NB = _N // _NS

_NH = 2
_G = 16
_SUP = 5
_SE = _SUP * _CH
_NSUP = _NCH // _SUP
_SUQ, _SUR = divmod(_NSUP, _NS)

@functools.lru_cache(maxsize=None)
def _mesh():
    return plsc.VectorSubcoreMesh(core_axis_name="c", subcore_axis_name="s",
                                  num_cores=_NC, num_subcores=_NS)


@functools.lru_cache(maxsize=None)
def _build_gather(ne):
    gq, gr = divmod(ne // _CH, _NW)
    @functools.partial(
        pl.kernel,
        out_type=(
            jax.ShapeDtypeStruct((ne, _W), jnp.bfloat16),
            jax.ShapeDtypeStruct((ne, _W), jnp.bfloat16),
        ),
        mesh=_mesh(),
        scratch_types=[
            pltpu.VMEM((_CH,), jnp.int32),
            pltpu.VMEM((_CH,), jnp.int32),
            pltpu.VMEM((_CH,), jnp.int32),
            pltpu.VMEM((_CH,), jnp.int32),
            pltpu.VMEM((_CH, _W), jnp.bfloat16),
            pltpu.VMEM((_CH, _W), jnp.bfloat16),
            pltpu.VMEM((_CH, _W), jnp.bfloat16),
            pltpu.VMEM((_CH, _W), jnp.bfloat16),
            pltpu.SemaphoreType.DMA,
            pltpu.SemaphoreType.DMA,
            pltpu.SemaphoreType.DMA,
            pltpu.SemaphoreType.DMA,
        ],
        compiler_params=pltpu.CompilerParams(use_tc_tiling_on_sc=False),
    )
    def gather_k(tab_hbm, src_hbm, dst_hbm, g1_hbm, g2_hbm,
                 idxs_a, idxd_a, idxs_b, idxd_b,
                 g1_a, g2_a, g1_b, g2_b,
                 sem_ga, sem_gb, sem_sa, sem_sb):
        cid = lax.axis_index("c")
        sid = lax.axis_index("s")
        wid = sid * _NC + cid
        nch = gq + jnp.where(wid < gr, 1, 0)
        base = gq * wid + jnp.minimum(wid, gr)

        def fire_gathers(c, idxs_v, idxd_v, g1_v, g2_v, sem_g):
            off = c * _CH
            pltpu.sync_copy(src_hbm.at[pl.ds(off, _CH)], idxs_v)
            pltpu.sync_copy(dst_hbm.at[pl.ds(off, _CH)], idxd_v)
            pltpu.async_copy(tab_hbm.at[idxs_v], g1_v, sem_g)
            pltpu.async_copy(tab_hbm.at[idxd_v], g2_v, sem_g)

        def wait_gathers(g1_v, g2_v, sem_g):
            pltpu.make_async_copy(g1_hbm.at[pl.ds(0, _CH)], g1_v, sem_g).wait()
            pltpu.make_async_copy(g2_hbm.at[pl.ds(0, _CH)], g2_v, sem_g).wait()

        def fire_stores(c, g1_v, g2_v, sem_s):
            off = c * _CH
            pltpu.async_copy(g1_v, g1_hbm.at[pl.ds(off, _CH)], sem_s)
            pltpu.async_copy(g2_v, g2_hbm.at[pl.ds(off, _CH)], sem_s)

        def wait_stores(g1_v, g2_v, sem_s):
            pltpu.make_async_copy(g1_hbm.at[pl.ds(0, _CH)], g1_v, sem_s).wait()
            pltpu.make_async_copy(g2_hbm.at[pl.ds(0, _CH)], g2_v, sem_s).wait()

        fire_gathers(base, idxs_a, idxd_a, g1_a, g2_a, sem_ga)

        def body(i, carry):
            c = base + i

            def step(idxs_n, idxd_n, g1_c, g2_c, g1_n, g2_n,
                     sem_gc, sem_gn, sem_sc, sem_sn):
                @pl.when(i >= 1)
                def _():
                    wait_stores(g1_n, g2_n, sem_sn)
                @pl.when(i + 1 < nch)
                def _():
                    fire_gathers(c + 1, idxs_n, idxd_n, g1_n, g2_n, sem_gn)
                wait_gathers(g1_c, g2_c, sem_gc)
                fire_stores(c, g1_c, g2_c, sem_sc)

            @pl.when((i % 2) == 0)
            def _():
                step(idxs_b, idxd_b, g1_a, g2_a, g1_b, g2_b,
                     sem_ga, sem_gb, sem_sa, sem_sb)

            @pl.when((i % 2) == 1)
            def _():
                step(idxs_a, idxd_a, g1_b, g2_b, g1_a, g2_a,
                     sem_gb, sem_ga, sem_sb, sem_sa)

            return carry

        lax.fori_loop(0, nch, body, 0)
        @pl.when((nch % 2) == 1)
        def _():
            wait_stores(g1_a, g2_a, sem_sa)

        @pl.when((nch % 2) == 0)
        def _():
            wait_stores(g1_b, g2_b, sem_sb)

    return gather_k


@functools.lru_cache(maxsize=None)
def _build_scatter(ne):
    suq, sur = divmod(ne // _SE, _NS)
    @functools.partial(
        pl.kernel,
        out_type=jax.ShapeDtypeStruct((_N, _W), jnp.float32),
        mesh=_mesh(),
        scratch_types=[
            pltpu.VMEM((_SUP, _CH), jnp.int32),
            pltpu.VMEM((_SUP, _CH), jnp.int32),
            pltpu.VMEM((_SE, _G), jnp.float32),
            pltpu.VMEM((_SE, _G), jnp.float32),
            pltpu.VMEM_SHARED((_N, _G), jnp.float32),
            pltpu.SemaphoreType.DMA,
            pltpu.SemaphoreType.DMA,
        ],
        compiler_params=pltpu.CompilerParams(use_tc_tiling_on_sc=False),
    )
    def scatter_k(dst2_hbm, msg_hbm, tab_hbm, out_hbm,
                  idx_a, idx_b, msg_a, msg_b, acc_sh, sem_l, sem_s):
        cid = lax.axis_index("c")
        sid = lax.axis_index("s")
        nsup = suq + jnp.where(sid < sur, 1, 0)
        sbase = suq * sid + jnp.minimum(sid, sur)

        def run_pass(goff):
            pltpu.sync_copy(
                tab_hbm.at[pl.ds(sid * _NB, _NB), pl.ds(goff, _G)],
                acc_sh.at[pl.ds(sid * _NB, _NB)])
            plsc.subcore_barrier()

            def load(s, idx_v, msg_v, sem):
                c1 = pltpu.async_copy(dst2_hbm.at[pl.ds(s * _SUP, _SUP)],
                                      idx_v, sem)
                c2 = pltpu.async_copy(
                    msg_hbm.at[pl.ds(s * _SE, _SE), pl.ds(goff, _G)],
                    msg_v, sem)
                return c1, c2

            c1, c2 = load(sbase, idx_a, msg_a, sem_l)
            c1.wait()
            c2.wait()

            def body(i, carry):
                def step(idx_c, msg_c, idx_n, msg_n):
                    descs = [
                        pltpu.async_copy(msg_c.at[pl.ds(j * _CH, _CH)],
                                         acc_sh.at[idx_c.at[j]],
                                         sem_s, add=True)
                        for j in range(_SUP)
                    ]
                    @pl.when(i + 1 < nsup)
                    def _():
                        n1, n2 = load(sbase + i + 1, idx_n, msg_n, sem_l)
                        n1.wait()
                        n2.wait()
                    for dsc in descs:
                        dsc.wait()

                @pl.when((i % 2) == 0)
                def _():
                    step(idx_a, msg_a, idx_b, msg_b)

                @pl.when((i % 2) == 1)
                def _():
                    step(idx_b, msg_b, idx_a, msg_a)

                return carry

            lax.fori_loop(0, nsup, body, 0)
            plsc.subcore_barrier()
            pltpu.sync_copy(
                acc_sh.at[pl.ds(sid * _NB, _NB)],
                out_hbm.at[pl.ds(sid * _NB, _NB), pl.ds(goff, _G)])
            plsc.subcore_barrier()

        @pl.when(cid == 0)
        def _():
            for g in range(4):
                run_pass(g * _G)

        @pl.when(cid == 1)
        def _():
            for g in range(3):
                run_pass(_S + g * _G)

    return scatter_k


_BT = 3200


def _tc_body(evt_ref, g1_ref, g2_ref,
             rw1_ref, rb1_ref, rw2_ref, rb2_ref,
             aw1a_ref, aw1b_ref, aw1c_ref, ab1_ref, aw2_ref, ab2_ref,
             mo_ref):
    x = evt_ref[0:1, :]
    y = evt_ref[1:2, :]
    z = evt_ref[2:3, :]
    d = jnp.sqrt(x * x + y * y + z * z)
    inv = 1.0 / jnp.maximum(d, 1e-8)

    theta = d * (np.pi / _CUT)
    s1 = jnp.sin(theta)
    c1 = jnp.cos(theta)
    cut = jnp.where(d < _CUT, 0.5 * (c1 + 1.0), 0.0)
    scale = inv * cut
    two_c = 2.0 * c1
    rows = []
    sk_prev = jnp.zeros_like(s1)
    sk = s1
    for _ in range(_R):
        rows.append(sk * scale)
        sk, sk_prev = two_c * sk - sk_prev, sk
    rbf_t = jnp.concatenate(rows, axis=0)

    h_t = jnp.dot(rw1_ref[...], rbf_t,
                  preferred_element_type=jnp.float32) + rb1_ref[...]
    h_t = h_t * jax.nn.sigmoid(h_t)
    filt_t = jnp.dot(rw2_ref[...], h_t,
                     preferred_element_type=jnp.float32) + rb2_ref[...]

    g1v = g1_ref[...].astype(jnp.float32)
    g2v = g2_ref[:, 0:_S].astype(jnp.float32)
    sj_t = g1v[:, 0:_S].T
    vj_t = g1v[:, _S:_S + _V3].T
    si_t = g2v.T
    ah_t = (jnp.dot(aw1a_ref[...], si_t, preferred_element_type=jnp.float32)
            + jnp.dot(aw1b_ref[...], sj_t, preferred_element_type=jnp.float32)
            + jnp.dot(aw1c_ref[...], rbf_t, preferred_element_type=jnp.float32)
            + ab1_ref[...])
    ah_t = ah_t * jax.nn.sigmoid(ah_t)
    att_t = jax.nn.sigmoid(
        jnp.dot(aw2_ref[...], ah_t, preferred_element_type=jnp.float32)
        + ab2_ref[...])

    sm_t = (att_t * filt_t[0:_S]) * sj_t
    avf_t = att_t * filt_t[_S:_S + 16]
    avg_t = att_t * filt_t[_S + 16:]
    mo_t = jnp.concatenate([
        sm_t,
        vj_t[0:16] * avf_t + avg_t * (y * inv),
        vj_t[16:32] * avf_t + avg_t * (z * inv),
        vj_t[32:48] * avf_t + avg_t * (x * inv),
        jnp.zeros((_W - _S - _V3, _BT), jnp.float32),
    ], axis=0)
    mo_ref[...] = mo_t.T


def _tc_edges(edge_vec, g1, g2, rw1, rb1, rw2, rb2, aw1, ab1, aw2, ab2):
    nblk = g1.shape[0] // _BT
    eb = lambda i: (i, 0)
    w = lambda i: (0, 0)
    return pl.pallas_call(
        _tc_body,
        grid=(nblk,),
        in_specs=[
            pl.BlockSpec((3, _BT), lambda i: (0, i)),
            pl.BlockSpec((_BT, _W), eb),
            pl.BlockSpec((_BT, _W), eb),
            pl.BlockSpec((_R, _R), w),
            pl.BlockSpec((_R, 1), w),
            pl.BlockSpec((96, _R), w),
            pl.BlockSpec((96, 1), w),
            pl.BlockSpec((_S, _S), w),
            pl.BlockSpec((_S, _S), w),
            pl.BlockSpec((_S, _R), w),
            pl.BlockSpec((_S, 1), w),
            pl.BlockSpec((1, _S), w),
            pl.BlockSpec((1, 1), w),
        ],
        out_specs=pl.BlockSpec((_BT, _W), eb),
        out_shape=jax.ShapeDtypeStruct((g1.shape[0], _W), jnp.float32),
        compiler_params=pltpu.CompilerParams(
            dimension_semantics=("arbitrary",),
        ),
    )(edge_vec.T, g1, g2, rw1.T, rb1.reshape(_R, 1), rw2.T, rb2.reshape(96, 1),
      aw1[:_S].T, aw1[_S:2 * _S].T, aw1[2 * _S:].T, ab1.reshape(_S, 1),
      aw2.T, ab2.reshape(1, 1))


def kernel(scalars, vectors, edge_index, edge_vec, rw1, rb1, rw2, rb2,
           aw1, ab1, aw2, ab2):
    src = edge_index[0]
    dst = edge_index[1]
    vec_t = vectors.transpose(0, 2, 1).reshape(_N, _V3)
    table = jnp.concatenate(
        [scalars, vec_t, jnp.zeros((_N, _W - _S - _V3), jnp.float32)], axis=1)
    table_bf = table.astype(jnp.bfloat16)

    nh = _E // _NH
    big = table
    mos = []
    for h in range(_NH):
        sl = slice(h * nh, (h + 1) * nh)
        g1, g2 = _build_gather(nh)(table_bf, src[sl], dst[sl])
        mos.append(_tc_edges(edge_vec[sl], g1, g2, rw1, rb1, rw2, rb2,
                             aw1, ab1, aw2, ab2))
    for h in range(_NH):
        sl = slice(h * nh, (h + 1) * nh)
        big = _build_scatter(nh)(dst[sl].reshape(nh // _CH, _CH), mos[h], big)

    out_scalars = big[:, 0:_S]
    out_vectors = big[:, _S:_S + _V3].reshape(_N, 3, 16).transpose(0, 2, 1)
    return out_scalars, out_vectors


# speedup vs baseline: 1.7614x; 1.7614x over previous
import functools

import numpy as np
import jax
import jax.numpy as jnp
from jax import lax
from jax.experimental import pallas as pl
from jax.experimental.pallas import tpu as pltpu
from jax.experimental.pallas import tpu_sc as plsc

_N = 50000
_E = 800000
_S = 64
_V3 = 48
_R = 32
_CUT = 10.0
_W = 128

_NC = 2
_NS = 16
_NW = _NC * _NS

_CH = 128
_NCH = _E // _CH
_GQ, _GR = divmod(_NCH, _NW)
_SQ, _SR = divmod(_NCH, _NS)
_NB = _N // _NS

_NH = 2
_G = 16
_SUP = 5
_SE = _SUP * _CH
_NSUP = _NCH // _SUP
_SUQ, _SUR = divmod(_NSUP, _NS)

@functools.lru_cache(maxsize=None)
def _mesh():
    return plsc.VectorSubcoreMesh(core_axis_name="c", subcore_axis_name="s",
                                  num_cores=_NC, num_subcores=_NS)


@functools.lru_cache(maxsize=None)
def _build_gather(ne):
    gq, gr = divmod(ne // _CH, _NW)
    @functools.partial(
        pl.kernel,
        out_type=(
            jax.ShapeDtypeStruct((ne, _W), jnp.float32),
            jax.ShapeDtypeStruct((ne, _W), jnp.float32),
        ),
        mesh=_mesh(),
        scratch_types=[
            pltpu.VMEM((_CH,), jnp.int32),
            pltpu.VMEM((_CH,), jnp.int32),
            pltpu.VMEM((_CH,), jnp.int32),
            pltpu.VMEM((_CH,), jnp.int32),
            pltpu.VMEM((_CH, _W), jnp.float32),
            pltpu.VMEM((_CH, _W), jnp.float32),
            pltpu.VMEM((_CH, _W), jnp.float32),
            pltpu.VMEM((_CH, _W), jnp.float32),
            pltpu.SemaphoreType.DMA,
            pltpu.SemaphoreType.DMA,
            pltpu.SemaphoreType.DMA,
            pltpu.SemaphoreType.DMA,
        ],
        compiler_params=pltpu.CompilerParams(use_tc_tiling_on_sc=False),
    )
    def gather_k(tab_hbm, src_hbm, dst_hbm, g1_hbm, g2_hbm,
                 idxs_a, idxd_a, idxs_b, idxd_b,
                 g1_a, g2_a, g1_b, g2_b,
                 sem_ga, sem_gb, sem_sa, sem_sb):
        cid = lax.axis_index("c")
        sid = lax.axis_index("s")
        wid = sid * _NC + cid
        nch = gq + jnp.where(wid < gr, 1, 0)
        base = gq * wid + jnp.minimum(wid, gr)

        def fire_gathers(c, idxs_v, idxd_v, g1_v, g2_v, sem_g):
            off = c * _CH
            pltpu.sync_copy(src_hbm.at[pl.ds(off, _CH)], idxs_v)
            pltpu.sync_copy(dst_hbm.at[pl.ds(off, _CH)], idxd_v)
            pltpu.async_copy(tab_hbm.at[idxs_v], g1_v, sem_g)
            pltpu.async_copy(tab_hbm.at[idxd_v], g2_v, sem_g)

        def wait_gathers(g1_v, g2_v, sem_g):
            pltpu.make_async_copy(g1_hbm.at[pl.ds(0, _CH)], g1_v, sem_g).wait()
            pltpu.make_async_copy(g2_hbm.at[pl.ds(0, _CH)], g2_v, sem_g).wait()

        def fire_stores(c, g1_v, g2_v, sem_s):
            off = c * _CH
            pltpu.async_copy(g1_v, g1_hbm.at[pl.ds(off, _CH)], sem_s)
            pltpu.async_copy(g2_v, g2_hbm.at[pl.ds(off, _CH)], sem_s)

        def wait_stores(g1_v, g2_v, sem_s):
            pltpu.make_async_copy(g1_hbm.at[pl.ds(0, _CH)], g1_v, sem_s).wait()
            pltpu.make_async_copy(g2_hbm.at[pl.ds(0, _CH)], g2_v, sem_s).wait()

        fire_gathers(base, idxs_a, idxd_a, g1_a, g2_a, sem_ga)

        def body(i, carry):
            c = base + i

            def step(idxs_n, idxd_n, g1_c, g2_c, g1_n, g2_n,
                     sem_gc, sem_gn, sem_sc, sem_sn):
                @pl.when(i >= 1)
                def _():
                    wait_stores(g1_n, g2_n, sem_sn)
                @pl.when(i + 1 < nch)
                def _():
                    fire_gathers(c + 1, idxs_n, idxd_n, g1_n, g2_n, sem_gn)
                wait_gathers(g1_c, g2_c, sem_gc)
                fire_stores(c, g1_c, g2_c, sem_sc)

            @pl.when((i % 2) == 0)
            def _():
                step(idxs_b, idxd_b, g1_a, g2_a, g1_b, g2_b,
                     sem_ga, sem_gb, sem_sa, sem_sb)

            @pl.when((i % 2) == 1)
            def _():
                step(idxs_a, idxd_a, g1_b, g2_b, g1_a, g2_a,
                     sem_gb, sem_ga, sem_sb, sem_sa)

            return carry

        lax.fori_loop(0, nch, body, 0)
        @pl.when((nch % 2) == 1)
        def _():
            wait_stores(g1_a, g2_a, sem_sa)

        @pl.when((nch % 2) == 0)
        def _():
            wait_stores(g1_b, g2_b, sem_sb)

    return gather_k


@functools.lru_cache(maxsize=None)
def _build_scatter(ne):
    suq, sur = divmod(ne // _SE, _NS)
    @functools.partial(
        pl.kernel,
        out_type=jax.ShapeDtypeStruct((_N, _W), jnp.float32),
        mesh=_mesh(),
        scratch_types=[
            pltpu.VMEM((_SUP, _CH), jnp.int32),
            pltpu.VMEM((_SUP, _CH), jnp.int32),
            pltpu.VMEM((_SE, _G), jnp.float32),
            pltpu.VMEM((_SE, _G), jnp.float32),
            pltpu.VMEM_SHARED((_N, _G), jnp.float32),
            pltpu.SemaphoreType.DMA,
            pltpu.SemaphoreType.DMA,
        ],
        compiler_params=pltpu.CompilerParams(use_tc_tiling_on_sc=False),
    )
    def scatter_k(dst2_hbm, msg_hbm, tab_hbm, out_hbm,
                  idx_a, idx_b, msg_a, msg_b, acc_sh, sem_l, sem_s):
        cid = lax.axis_index("c")
        sid = lax.axis_index("s")
        nsup = suq + jnp.where(sid < sur, 1, 0)
        sbase = suq * sid + jnp.minimum(sid, sur)

        def run_pass(goff):
            pltpu.sync_copy(
                tab_hbm.at[pl.ds(sid * _NB, _NB), pl.ds(goff, _G)],
                acc_sh.at[pl.ds(sid * _NB, _NB)])
            plsc.subcore_barrier()

            def load(s, idx_v, msg_v, sem):
                c1 = pltpu.async_copy(dst2_hbm.at[pl.ds(s * _SUP, _SUP)],
                                      idx_v, sem)
                c2 = pltpu.async_copy(
                    msg_hbm.at[pl.ds(s * _SE, _SE), pl.ds(goff, _G)],
                    msg_v, sem)
                return c1, c2

            c1, c2 = load(sbase, idx_a, msg_a, sem_l)
            c1.wait()
            c2.wait()

            def body(i, carry):
                def step(idx_c, msg_c, idx_n, msg_n):
                    descs = [
                        pltpu.async_copy(msg_c.at[pl.ds(j * _CH, _CH)],
                                         acc_sh.at[idx_c.at[j]],
                                         sem_s, add=True)
                        for j in range(_SUP)
                    ]
                    @pl.when(i + 1 < nsup)
                    def _():
                        n1, n2 = load(sbase + i + 1, idx_n, msg_n, sem_l)
                        n1.wait()
                        n2.wait()
                    for dsc in descs:
                        dsc.wait()

                @pl.when((i % 2) == 0)
                def _():
                    step(idx_a, msg_a, idx_b, msg_b)

                @pl.when((i % 2) == 1)
                def _():
                    step(idx_b, msg_b, idx_a, msg_a)

                return carry

            lax.fori_loop(0, nsup, body, 0)
            plsc.subcore_barrier()
            pltpu.sync_copy(
                acc_sh.at[pl.ds(sid * _NB, _NB)],
                out_hbm.at[pl.ds(sid * _NB, _NB), pl.ds(goff, _G)])
            plsc.subcore_barrier()

        @pl.when(cid == 0)
        def _():
            for g in range(4):
                run_pass(g * _G)

        @pl.when(cid == 1)
        def _():
            for g in range(3):
                run_pass(_S + g * _G)

    return scatter_k


_BT = 3200


def _tc_body(evt_ref, g1_ref, g2_ref,
             rw1_ref, rb1_ref, rw2_ref, rb2_ref,
             aw1a_ref, aw1b_ref, aw1c_ref, ab1_ref, aw2_ref, ab2_ref,
             mo_ref):
    x = evt_ref[0:1, :]
    y = evt_ref[1:2, :]
    z = evt_ref[2:3, :]
    d = jnp.sqrt(x * x + y * y + z * z)
    inv = 1.0 / jnp.maximum(d, 1e-8)

    theta = d * (np.pi / _CUT)
    s1 = jnp.sin(theta)
    c1 = jnp.cos(theta)
    cut = jnp.where(d < _CUT, 0.5 * (c1 + 1.0), 0.0)
    scale = inv * cut
    two_c = 2.0 * c1
    rows = []
    sk_prev = jnp.zeros_like(s1)
    sk = s1
    for _ in range(_R):
        rows.append(sk * scale)
        sk, sk_prev = two_c * sk - sk_prev, sk
    rbf_t = jnp.concatenate(rows, axis=0)

    h_t = jnp.dot(rw1_ref[...], rbf_t,
                  preferred_element_type=jnp.float32) + rb1_ref[...]
    h_t = h_t * jax.nn.sigmoid(h_t)
    filt_t = jnp.dot(rw2_ref[...], h_t,
                     preferred_element_type=jnp.float32) + rb2_ref[...]

    sj_t = g1_ref[:, 0:_S].T
    vj_t = g1_ref[:, _S:_S + _V3].T
    si_t = g2_ref[:, 0:_S].T
    ah_t = (jnp.dot(aw1a_ref[...], si_t, preferred_element_type=jnp.float32)
            + jnp.dot(aw1b_ref[...], sj_t, preferred_element_type=jnp.float32)
            + jnp.dot(aw1c_ref[...], rbf_t, preferred_element_type=jnp.float32)
            + ab1_ref[...])
    ah_t = ah_t * jax.nn.sigmoid(ah_t)
    att_t = jax.nn.sigmoid(
        jnp.dot(aw2_ref[...], ah_t, preferred_element_type=jnp.float32)
        + ab2_ref[...])

    sm_t = (att_t * filt_t[0:_S]) * sj_t
    avf_t = att_t * filt_t[_S:_S + 16]
    avg_t = att_t * filt_t[_S + 16:]
    mo_t = jnp.concatenate([
        sm_t,
        vj_t[0:16] * avf_t + avg_t * (y * inv),
        vj_t[16:32] * avf_t + avg_t * (z * inv),
        vj_t[32:48] * avf_t + avg_t * (x * inv),
        jnp.zeros((_W - _S - _V3, _BT), jnp.float32),
    ], axis=0)
    mo_ref[...] = mo_t.T


def _tc_edges(edge_vec, g1, g2, rw1, rb1, rw2, rb2, aw1, ab1, aw2, ab2):
    nblk = g1.shape[0] // _BT
    eb = lambda i: (i, 0)
    w = lambda i: (0, 0)
    return pl.pallas_call(
        _tc_body,
        grid=(nblk,),
        in_specs=[
            pl.BlockSpec((3, _BT), lambda i: (0, i)),
            pl.BlockSpec((_BT, _W), eb),
            pl.BlockSpec((_BT, _W), eb),
            pl.BlockSpec((_R, _R), w),
            pl.BlockSpec((_R, 1), w),
            pl.BlockSpec((96, _R), w),
            pl.BlockSpec((96, 1), w),
            pl.BlockSpec((_S, _S), w),
            pl.BlockSpec((_S, _S), w),
            pl.BlockSpec((_S, _R), w),
            pl.BlockSpec((_S, 1), w),
            pl.BlockSpec((1, _S), w),
            pl.BlockSpec((1, 1), w),
        ],
        out_specs=pl.BlockSpec((_BT, _W), eb),
        out_shape=jax.ShapeDtypeStruct((g1.shape[0], _W), jnp.float32),
        compiler_params=pltpu.CompilerParams(
            dimension_semantics=("arbitrary",),
        ),
    )(edge_vec.T, g1, g2, rw1.T, rb1.reshape(_R, 1), rw2.T, rb2.reshape(96, 1),
      aw1[:_S].T, aw1[_S:2 * _S].T, aw1[2 * _S:].T, ab1.reshape(_S, 1),
      aw2.T, ab2.reshape(1, 1))


def kernel(scalars, vectors, edge_index, edge_vec, rw1, rb1, rw2, rb2,
           aw1, ab1, aw2, ab2):
    src = edge_index[0]
    dst = edge_index[1]
    vec_t = vectors.transpose(0, 2, 1).reshape(_N, _V3)
    table = jnp.concatenate(
        [scalars, vec_t, jnp.zeros((_N, _W - _S - _V3), jnp.float32)], axis=1)

    nh = _E // _NH
    big = table
    mos = []
    for h in range(_NH):
        sl = slice(h * nh, (h + 1) * nh)
        g1, g2 = _build_gather(nh)(table, src[sl], dst[sl])
        mos.append(_tc_edges(edge_vec[sl], g1, g2, rw1, rb1, rw2, rb2,
                             aw1, ab1, aw2, ab2))
    for h in range(_NH):
        sl = slice(h * nh, (h + 1) * nh)
        big = _build_scatter(nh)(dst[sl].reshape(nh // _CH, _CH), mos[h], big)

    out_scalars = big[:, 0:_S]
    out_vectors = big[:, _S:_S + _V3].reshape(_N, 3, 16).transpose(0, 2, 1)
    return out_scalars, out_vectors
